# Initial kernel scaffold; baseline (speedup 1.0000x reference)
import jax
import jax.numpy as jnp
from jax.experimental import pallas as pl


def kernel(x, expert_weights, mlp1_weight, mlp1_bias, mlp2_weight, mlp2_bias, expert_indices):
    raise NotImplementedError("write your pallas kernel here")


# speedup vs baseline: 1.4729x; 1.4729x over previous
import jax
import jax.numpy as jnp
from jax.experimental import pallas as pl

ALPHA, LIMIT = 1.702, 7.0


def _moe_body(xt_ref, w1_ref, b1_ref, w2_ref, b2_ref, idx_ref, wgt_ref, out_ref):
    e = pl.program_id(0)
    H = xt_ref.shape[0]

    @pl.when(e == 0)
    def _init():
        out_ref[...] = jnp.zeros_like(out_ref)

    idx = idx_ref[0]
    wgt = wgt_ref[0].astype(jnp.float32)
    c = jnp.sum(jnp.where(idx == e, wgt, 0.0), axis=0, keepdims=True)

    xt = xt_ref[...]
    x_glu = jax.lax.dot_general(
        w1_ref[0, :, :H], xt, (((1,), (0,)), ((), ())),
        preferred_element_type=jnp.float32)
    x_lin = jax.lax.dot_general(
        w1_ref[0, :, H:], xt, (((1,), (0,)), ((), ())),
        preferred_element_type=jnp.float32)
    b1 = b1_ref[0].astype(jnp.float32)
    x_glu = x_glu + b1[:, 0:1]
    x_lin = x_lin + b1[:, 1:2]
    x_glu = x_glu.astype(jnp.bfloat16).astype(jnp.float32)
    x_lin = x_lin.astype(jnp.bfloat16).astype(jnp.float32)

    x_glu = jnp.minimum(x_glu, LIMIT)
    x_lin = jnp.clip(x_lin, -LIMIT, LIMIT)
    act = (x_glu * jax.nn.sigmoid(ALPHA * x_glu)) * (x_lin + 1.0)
    act = act.astype(jnp.bfloat16)

    t2 = jax.lax.dot_general(
        w2_ref[0], act, (((1,), (0,)), ((), ())),
        preferred_element_type=jnp.float32)
    t2 = t2 + b2_ref[0].astype(jnp.float32)

    out_ref[...] += t2 * c


def kernel(x, expert_weights, mlp1_weight, mlp1_bias, mlp2_weight, mlp2_bias,
           expert_indices):
    T, H = x.shape
    E, two_i, _ = mlp1_weight.shape
    K = expert_indices.shape[1]
    I = two_i // 2

    xt = x.T
    w1v = mlp1_weight.reshape(E, I, 2 * H)
    b1v = mlp1_bias.reshape(E, I, 2)
    b2c = mlp2_bias[:, :, None]
    idxT = expert_indices.astype(jnp.int32).T[None]
    wgtT = expert_weights.T[None]

    out = pl.pallas_call(
        _moe_body,
        grid=(E,),
        in_specs=[
            pl.BlockSpec((H, T), lambda e: (0, 0)),
            pl.BlockSpec((1, I, 2 * H), lambda e: (e, 0, 0)),
            pl.BlockSpec((1, I, 2), lambda e: (e, 0, 0)),
            pl.BlockSpec((1, H, I), lambda e: (e, 0, 0)),
            pl.BlockSpec((1, H, 1), lambda e: (e, 0, 0)),
            pl.BlockSpec((1, K, T), lambda e: (0, 0, 0)),
            pl.BlockSpec((1, K, T), lambda e: (0, 0, 0)),
        ],
        out_specs=pl.BlockSpec((H, T), lambda e: (0, 0)),
        out_shape=jax.ShapeDtypeStruct((H, T), jnp.float32),
    )(xt, w1v, b1v, mlp2_weight, b2c, idxT, wgtT)

    return out.T.astype(x.dtype)


# speedup vs baseline: 1.6027x; 1.0881x over previous
import jax
import jax.numpy as jnp
from jax.experimental import pallas as pl

ALPHA, LIMIT = 1.702, 7.0


def _moe_body(x_ref, w1_ref, b1_ref, w2_ref, b2_ref, idx_ref, wgt_ref, out_ref):
    e = pl.program_id(0)
    H = x_ref.shape[1]

    @pl.when(e == 0)
    def _init():
        out_ref[...] = jnp.zeros_like(out_ref)

    idx = idx_ref[0]
    wgt = wgt_ref[0].astype(jnp.float32)
    c = jnp.sum(jnp.where(idx == e, wgt, 0.0), axis=1, keepdims=True)

    x = x_ref[...]
    x_glu = jax.lax.dot_general(
        x, w1_ref[0, :, :H], (((1,), (1,)), ((), ())),
        preferred_element_type=jnp.float32)
    x_lin = jax.lax.dot_general(
        x, w1_ref[0, :, H:], (((1,), (1,)), ((), ())),
        preferred_element_type=jnp.float32)
    b1 = b1_ref[0].astype(jnp.float32)
    x_glu = x_glu + b1[0:1, :]
    x_lin = x_lin + b1[1:2, :]
    x_glu = x_glu.astype(jnp.bfloat16).astype(jnp.float32)
    x_lin = x_lin.astype(jnp.bfloat16).astype(jnp.float32)

    x_glu = jnp.minimum(x_glu, LIMIT)
    x_lin = jnp.clip(x_lin, -LIMIT, LIMIT)
    act = (x_glu * jax.nn.sigmoid(ALPHA * x_glu)) * (x_lin + 1.0)
    act = act.astype(jnp.bfloat16)

    t2 = jax.lax.dot_general(
        act, w2_ref[0], (((1,), (1,)), ((), ())),
        preferred_element_type=jnp.float32)
    t2 = t2 + b2_ref[0].astype(jnp.float32)

    out_ref[...] += t2 * c


def kernel(x, expert_weights, mlp1_weight, mlp1_bias, mlp2_weight, mlp2_bias,
           expert_indices):
    T, H = x.shape
    E, two_i, _ = mlp1_weight.shape
    K = expert_indices.shape[1]
    I = two_i // 2

    w1v = mlp1_weight.reshape(E, I, 2 * H)
    b1v = mlp1_bias.reshape(E, I, 2).transpose(0, 2, 1)
    b2r = mlp2_bias[:, None, :]
    idx3 = expert_indices.astype(jnp.int32)[None]
    wgt3 = expert_weights[None]

    out = pl.pallas_call(
        _moe_body,
        grid=(E,),
        in_specs=[
            pl.BlockSpec((T, H), lambda e: (0, 0)),
            pl.BlockSpec((1, I, 2 * H), lambda e: (e, 0, 0)),
            pl.BlockSpec((1, 2, I), lambda e: (e, 0, 0)),
            pl.BlockSpec((1, H, I), lambda e: (e, 0, 0)),
            pl.BlockSpec((1, 1, H), lambda e: (e, 0, 0)),
            pl.BlockSpec((1, T, K), lambda e: (0, 0, 0)),
            pl.BlockSpec((1, T, K), lambda e: (0, 0, 0)),
        ],
        out_specs=pl.BlockSpec((T, H), lambda e: (0, 0)),
        out_shape=jax.ShapeDtypeStruct((T, H), jnp.float32),
    )(x, w1v, b1v, mlp2_weight, b2r, idx3, wgt3)

    return out.astype(x.dtype)
